# baseline (speedup 1.0000x reference)
import functools

import jax
import jax.numpy as jnp
from jax import lax
from jax.experimental import pallas as pl
from jax.experimental.pallas import tpu as pltpu
from jax.experimental.pallas import tpu_sc as plsc


def _matmul_body(h_ref, w_ref, o_ref):
    o_ref[...] = jnp.dot(h_ref[...], w_ref[0],
                         preferred_element_type=jnp.float32)


def _combine_body(self_ref, m_ref, deg_ref, b_ref, o_ref):
    inv = (1.0 / jnp.maximum(deg_ref[:, 0], 1.0))[:, None]
    o_ref[...] = self_ref[...] + b_ref[0] + m_ref[...] * inv


def kernel(h, edge_index, edge_type, num_nodes, weight, W_self, b_self):
    N, D = h.shape
    R = weight.shape[0]
    E = edge_type.shape[0]

    NS, L = 16, 16
    NW = NS
    CHUNK = 96
    ROWS_PT = (N + NS - 1) // NS + 1
    ROWS_PT = (ROWS_PT + 7) // 8 * 8
    NPAD = ROWS_PT * NS
    TRASH = N

    NCH = -(-E // (NW * CHUNK))
    NCH = (NCH + 3) // 4 * 4
    NCHA = NCH + 4
    EPAD = NW * NCH * CHUNK

    src = edge_index[0]
    dst = edge_index[1]
    npad_e = EPAD - E
    pad_g = (jnp.arange(npad_e, dtype=jnp.int32) * 64) % (R * N)
    pad_d = TRASH + (jnp.arange(npad_e, dtype=jnp.int32) % (NPAD - N))
    gidx = jnp.concatenate([edge_type * N + src, pad_g]).reshape(NW, NCH, CHUNK)
    didx = jnp.concatenate([dst, pad_d]).reshape(NW, NCH, CHUNK)
    gidx = jnp.concatenate(
        [gidx, jnp.zeros((NW, NCHA - NCH, CHUNK), jnp.int32)], axis=1)
    didx = jnp.concatenate(
        [didx, jnp.full((NW, NCHA - NCH, CHUNK), TRASH, jnp.int32)], axis=1)
    idx_hbm_arr = jnp.stack([gidx, didx], axis=2)
    w_cat = jnp.concatenate([weight, W_self[None]], axis=0)

    BN = 1000
    NB = N // BN
    hw_all = pl.pallas_call(
        _matmul_body,
        grid=(NB, R + 1),
        in_specs=[
            pl.BlockSpec((BN, D), lambda j, r: (j, 0)),
            pl.BlockSpec((1, D, D), lambda j, r: (r, 0, 0)),
        ],
        out_specs=pl.BlockSpec((BN, D), lambda j, r: (r * NB + j, 0)),
        out_shape=jax.ShapeDtypeStruct(((R + 1) * N, D), jnp.float32),
    )(h, w_cat)

    mesh = plsc.VectorSubcoreMesh(
        core_axis_name="c", subcore_axis_name="s", num_cores=1)

    @functools.partial(
        pl.kernel,
        mesh=mesh,
        compiler_params=pltpu.CompilerParams(use_tc_tiling_on_sc=False),
        out_type=[
            jax.ShapeDtypeStruct((NPAD, D), jnp.float32),
            jax.ShapeDtypeStruct((NPAD, D), jnp.float32),
        ],
        scratch_types=[
            pltpu.VMEM((2, 2, CHUNK), jnp.int32),
            pltpu.VMEM((2, 2, CHUNK), jnp.int32),
            pltpu.VMEM((CHUNK, D), jnp.float32),
            pltpu.VMEM((CHUNK, D), jnp.float32),
            pltpu.VMEM_SHARED((NPAD, D), jnp.float32),
            pltpu.SemaphoreType.DMA,
            pltpu.SemaphoreType.DMA,
            pltpu.SemaphoreType.DMA,
            pltpu.SemaphoreType.DMA,
        ],
    )
    def sc_scatter(hw_hbm, idx_hbm, m_out, deg_out,
                   slot0, slot1, buf_a, buf_b, m_sh,
                   sem_a, sem_b, sem_i0, sem_i1):
        sid = lax.axis_index("s")
        wid = sid
        base = sid * ROWS_PT

        zeros16 = jnp.zeros((L,), jnp.float32)
        ones16 = jnp.ones((L,), jnp.float32)

        def _fill(buf, val16):
            def _row(i, carry):
                for t in range(D // L):
                    buf[i, pl.ds(t * L, L)] = val16
                return carry
            lax.fori_loop(0, CHUNK, _row, 0)

        _fill(buf_a, zeros16)

        nfull = ROWS_PT // CHUNK
        tail = ROWS_PT - nfull * CHUNK
        nslices = nfull + (1 if tail else 0)

        def _zero_acc():
            for k in range(nfull):
                pltpu.sync_copy(buf_a, m_sh.at[pl.ds(base + k * CHUNK, CHUNK)])
            if tail:
                pltpu.sync_copy(buf_a.at[pl.ds(0, tail)],
                                m_sh.at[pl.ds(base + nfull * CHUNK, tail)])

        def _prime():
            pltpu.async_copy(idx_hbm.at[wid, pl.ds(0, 2)], slot0, sem_i0)
            pltpu.async_copy(idx_hbm.at[wid, pl.ds(2, 2)], slot1, sem_i1)

        def _slot_wait(slot, sem_i):
            pltpu.make_async_copy(
                idx_hbm.at[wid, pl.ds(0, 2)], slot, sem_i).wait()

        def _drain():
            _slot_wait(slot0, sem_i0)
            _slot_wait(slot1, sem_i1)

        _zero_acc()
        plsc.subcore_barrier()
        _prime()

        def _half(c_next, slot, sem_i):
            _slot_wait(slot, sem_i)
            cp_a = pltpu.async_copy(hw_hbm.at[slot.at[0, 0]], buf_a, sem_a)
            cp_b = pltpu.async_copy(hw_hbm.at[slot.at[1, 0]], buf_b, sem_b)
            cp_a.wait()
            pltpu.sync_copy(buf_a, m_sh.at[slot.at[0, 1]], add=True)
            cp_b.wait()
            pltpu.sync_copy(buf_b, m_sh.at[slot.at[1, 1]], add=True)
            pltpu.async_copy(idx_hbm.at[wid, pl.ds(c_next, 2)], slot, sem_i)

        def _body(i, carry):
            c0 = i * 4
            _half(c0 + 4, slot0, sem_i0)
            _half(c0 + 6, slot1, sem_i1)
            return carry

        lax.fori_loop(0, NCH // 4, _body, 0)
        _drain()
        plsc.subcore_barrier()

        def _writeout(dst_hbm):
            for k in range(nslices):
                off = base + k * CHUNK
                rows = CHUNK if k < nfull else tail
                pltpu.sync_copy(m_sh.at[pl.ds(off, rows)],
                                buf_a.at[pl.ds(0, rows)])
                pltpu.sync_copy(buf_a.at[pl.ds(0, rows)],
                                dst_hbm.at[pl.ds(off, rows)])

        _writeout(m_out)
        plsc.subcore_barrier()

        _fill(buf_a, zeros16)
        _zero_acc()
        _fill(buf_b, ones16)
        plsc.subcore_barrier()
        _prime()

        def _dhalf(c_next, slot, sem_i):
            _slot_wait(slot, sem_i)
            pltpu.sync_copy(buf_b, m_sh.at[slot.at[0, 1]], add=True)
            pltpu.sync_copy(buf_b, m_sh.at[slot.at[1, 1]], add=True)
            pltpu.async_copy(idx_hbm.at[wid, pl.ds(c_next, 2)], slot, sem_i)

        def _dbody(i, carry):
            c0 = i * 4
            _dhalf(c0 + 4, slot0, sem_i0)
            _dhalf(c0 + 6, slot1, sem_i1)
            return carry

        lax.fori_loop(0, NCH // 4, _dbody, 0)
        _drain()
        plsc.subcore_barrier()
        _writeout(deg_out)

    m_sum, deg_sum = sc_scatter(hw_all, idx_hbm_arr)

    out = pl.pallas_call(
        _combine_body,
        grid=(NB,),
        in_specs=[
            pl.BlockSpec((BN, D), lambda j: (R * NB + j, 0)),
            pl.BlockSpec((BN, D), lambda j: (j, 0)),
            pl.BlockSpec((BN, D), lambda j: (j, 0)),
            pl.BlockSpec((1, D), lambda j: (0, 0)),
        ],
        out_specs=pl.BlockSpec((BN, D), lambda j: (j, 0)),
        out_shape=jax.ShapeDtypeStruct((N, D), jnp.float32),
    )(hw_all, m_sum, deg_sum, b_self.reshape(1, D))
    return out


# speedup vs baseline: 1.1586x; 1.1586x over previous
import functools

import jax
import jax.numpy as jnp
from jax import lax
from jax.experimental import pallas as pl
from jax.experimental.pallas import tpu as pltpu
from jax.experimental.pallas import tpu_sc as plsc


def _matmul_body(h_ref, w_ref, o_ref):
    o_ref[...] = jnp.dot(h_ref[...], w_ref[0],
                         preferred_element_type=jnp.float32)


def _combine_body(self_ref, m_ref, deg_ref, b_ref, o_ref):
    inv = (1.0 / jnp.maximum(deg_ref[:, 0], 1.0))[:, None]
    o_ref[...] = self_ref[...] + b_ref[0] + m_ref[...] * inv


def kernel(h, edge_index, edge_type, num_nodes, weight, W_self, b_self):
    N, D = h.shape
    R = weight.shape[0]
    E = edge_type.shape[0]

    NS, L = 16, 16
    NW = NS
    CHUNK = 96
    ROWS_PT = (N + NS - 1) // NS + 1
    ROWS_PT = (ROWS_PT + 7) // 8 * 8
    NPAD = ROWS_PT * NS
    TRASH = N

    NCH = -(-E // (NW * CHUNK))
    NCH = (NCH + 3) // 4 * 4
    NCHA = NCH + 4
    EPAD = NW * NCH * CHUNK

    src = edge_index[0]
    dst = edge_index[1]
    npad_e = EPAD - E
    pad_g = (jnp.arange(npad_e, dtype=jnp.int32) * 64) % (R * N)
    pad_d = TRASH + (jnp.arange(npad_e, dtype=jnp.int32) % (NPAD - N))
    gidx = jnp.concatenate([edge_type * N + src, pad_g]).reshape(NW, NCH, CHUNK)
    didx = jnp.concatenate([dst, pad_d]).reshape(NW, NCH, CHUNK)
    gidx = jnp.concatenate(
        [gidx, jnp.zeros((NW, NCHA - NCH, CHUNK), jnp.int32)], axis=1)
    didx = jnp.concatenate(
        [didx, jnp.full((NW, NCHA - NCH, CHUNK), TRASH, jnp.int32)], axis=1)
    idx_hbm_arr = jnp.stack([gidx, didx], axis=2)
    w_cat = jnp.concatenate([weight, W_self[None]], axis=0)

    BN = 1000
    NB = N // BN
    hw_all = pl.pallas_call(
        _matmul_body,
        grid=(NB, R + 1),
        in_specs=[
            pl.BlockSpec((BN, D), lambda j, r: (j, 0)),
            pl.BlockSpec((1, D, D), lambda j, r: (r, 0, 0)),
        ],
        out_specs=pl.BlockSpec((BN, D), lambda j, r: (r * NB + j, 0)),
        out_shape=jax.ShapeDtypeStruct(((R + 1) * N, D), jnp.float32),
    )(h, w_cat)

    mesh = plsc.VectorSubcoreMesh(
        core_axis_name="c", subcore_axis_name="s", num_cores=1)

    @functools.partial(
        pl.kernel,
        mesh=mesh,
        compiler_params=pltpu.CompilerParams(use_tc_tiling_on_sc=False),
        out_type=[
            jax.ShapeDtypeStruct((NPAD, D), jnp.float32),
            jax.ShapeDtypeStruct((NPAD, L), jnp.float32),
        ],
        scratch_types=[
            pltpu.VMEM((2, 2, CHUNK), jnp.int32),
            pltpu.VMEM((2, 2, CHUNK), jnp.int32),
            pltpu.VMEM((CHUNK, D), jnp.float32),
            pltpu.VMEM((CHUNK, D), jnp.float32),
            pltpu.VMEM((CHUNK, L), jnp.float32),
            pltpu.VMEM_SHARED((NPAD, D), jnp.float32),
            pltpu.VMEM_SHARED((NPAD, L), jnp.float32),
            pltpu.SemaphoreType.DMA,
            pltpu.SemaphoreType.DMA,
            pltpu.SemaphoreType.DMA,
            pltpu.SemaphoreType.DMA,
        ],
    )
    def sc_scatter(hw_hbm, idx_hbm, m_out, deg_out,
                   slot0, slot1, buf_a, buf_b, one16_v, m_sh, deg_sh,
                   sem_a, sem_b, sem_i0, sem_i1):
        sid = lax.axis_index("s")
        wid = sid
        base = sid * ROWS_PT

        zeros16 = jnp.zeros((L,), jnp.float32)
        ones16 = jnp.ones((L,), jnp.float32)

        def _fill(buf, val16):
            def _row(i, carry):
                for t in range(D // L):
                    buf[i, pl.ds(t * L, L)] = val16
                return carry
            lax.fori_loop(0, CHUNK, _row, 0)

        _fill(buf_a, zeros16)

        nfull = ROWS_PT // CHUNK
        tail = ROWS_PT - nfull * CHUNK
        nslices = nfull + (1 if tail else 0)

        def _zero_acc():
            for k in range(nfull):
                pltpu.sync_copy(buf_a, m_sh.at[pl.ds(base + k * CHUNK, CHUNK)])
            if tail:
                pltpu.sync_copy(buf_a.at[pl.ds(0, tail)],
                                m_sh.at[pl.ds(base + nfull * CHUNK, tail)])

        def _prime():
            pltpu.async_copy(idx_hbm.at[wid, pl.ds(0, 2)], slot0, sem_i0)
            pltpu.async_copy(idx_hbm.at[wid, pl.ds(2, 2)], slot1, sem_i1)

        def _slot_wait(slot, sem_i):
            pltpu.make_async_copy(
                idx_hbm.at[wid, pl.ds(0, 2)], slot, sem_i).wait()

        def _drain():
            _slot_wait(slot0, sem_i0)
            _slot_wait(slot1, sem_i1)

        def _fill16(val16):
            def _row(i, carry):
                one16_v[i] = val16
                return carry
            lax.fori_loop(0, CHUNK, _row, 0)

        _fill16(zeros16)
        _zero_acc()
        iota16 = lax.iota(jnp.int32, L)
        for k in range(nslices):
            for t in range(CHUNK // L):
                j0 = k * CHUNK + t * L
                vals = jnp.minimum(iota16 + j0, ROWS_PT - 1) + base
                slot0[0, 0, pl.ds(t * L, L)] = vals
            pltpu.sync_copy(one16_v, deg_sh.at[slot0.at[0, 0]])
        plsc.subcore_barrier()
        _prime()

        def _half(c_next, slot, sem_i):
            _slot_wait(slot, sem_i)
            cp_a = pltpu.async_copy(hw_hbm.at[slot.at[0, 0]], buf_a, sem_a)
            cp_b = pltpu.async_copy(hw_hbm.at[slot.at[1, 0]], buf_b, sem_b)
            cp_a.wait()
            pltpu.sync_copy(buf_a, m_sh.at[slot.at[0, 1]], add=True)
            cp_b.wait()
            pltpu.sync_copy(buf_b, m_sh.at[slot.at[1, 1]], add=True)
            pltpu.async_copy(idx_hbm.at[wid, pl.ds(c_next, 2)], slot, sem_i)

        def _body(i, carry):
            c0 = i * 4
            _half(c0 + 4, slot0, sem_i0)
            _half(c0 + 6, slot1, sem_i1)
            return carry

        lax.fori_loop(0, NCH // 4, _body, 0)
        _drain()
        plsc.subcore_barrier()

        def _writeout(dst_hbm):
            for k in range(nslices):
                off = base + k * CHUNK
                rows = CHUNK if k < nfull else tail
                pltpu.sync_copy(m_sh.at[pl.ds(off, rows)],
                                buf_a.at[pl.ds(0, rows)])
                pltpu.sync_copy(buf_a.at[pl.ds(0, rows)],
                                dst_hbm.at[pl.ds(off, rows)])

        _writeout(m_out)
        plsc.subcore_barrier()

        _fill16(ones16)
        plsc.subcore_barrier()
        _prime()

        def _dhalf(c_next, slot, sem_i):
            _slot_wait(slot, sem_i)
            pltpu.sync_copy(one16_v, deg_sh.at[slot.at[0, 1]], add=True)
            pltpu.sync_copy(one16_v, deg_sh.at[slot.at[1, 1]], add=True)
            pltpu.async_copy(idx_hbm.at[wid, pl.ds(c_next, 2)], slot, sem_i)

        def _dbody(i, carry):
            c0 = i * 4
            _dhalf(c0 + 4, slot0, sem_i0)
            _dhalf(c0 + 6, slot1, sem_i1)
            return carry

        lax.fori_loop(0, NCH // 4, _dbody, 0)
        _drain()
        plsc.subcore_barrier()
        for k in range(nslices):
            off = base + k * CHUNK
            rows = CHUNK if k < nfull else tail
            for t in range(CHUNK // L):
                j0 = k * CHUNK + t * L
                vals = jnp.minimum(iota16 + j0, ROWS_PT - 1) + base
                slot0[0, 0, pl.ds(t * L, L)] = vals
            pltpu.sync_copy(deg_sh.at[slot0.at[0, 0]], one16_v)
            pltpu.sync_copy(one16_v.at[pl.ds(0, rows)],
                            deg_out.at[pl.ds(off, rows)])

    m_sum, deg_sum = sc_scatter(hw_all, idx_hbm_arr)

    out = pl.pallas_call(
        _combine_body,
        grid=(NB,),
        in_specs=[
            pl.BlockSpec((BN, D), lambda j: (R * NB + j, 0)),
            pl.BlockSpec((BN, D), lambda j: (j, 0)),
            pl.BlockSpec((BN, L), lambda j: (j, 0)),
            pl.BlockSpec((1, D), lambda j: (0, 0)),
        ],
        out_specs=pl.BlockSpec((BN, D), lambda j: (j, 0)),
        out_shape=jax.ShapeDtypeStruct((N, D), jnp.float32),
    )(hw_all, m_sum, deg_sum, b_self.reshape(1, D))
    return out
